# Initial kernel scaffold; baseline (speedup 1.0000x reference)
import jax
import jax.numpy as jnp
from jax.experimental import pallas as pl


def kernel(x, W):
    raise NotImplementedError("write your pallas kernel here")


# speedup vs baseline: 2.1197x; 2.1197x over previous
import functools

import jax
import jax.numpy as jnp
from jax.experimental import pallas as pl

_K = 8
_ROW_BLOCK = 16


def _topk_body(x_ref, w_ref, out_ref):
    x = x_ref[...]
    iota = jax.lax.broadcasted_iota(jnp.int32, x.shape, 1)
    k_hot = jnp.zeros_like(x)
    big = jnp.int32(2**30)
    for _ in range(_K):
        m = jnp.max(x, axis=1, keepdims=True)
        hit = x == m
        idx = jnp.min(jnp.where(hit, iota, big), axis=1, keepdims=True)
        onehot = iota == idx
        k_hot = k_hot + onehot.astype(x.dtype)
        x = jnp.where(onehot, -jnp.inf, x)
    out_ref[...] = jax.lax.dot_general(
        k_hot, w_ref[...], (((1,), (1,)), ((), ())),
        preferred_element_type=jnp.float32)


@jax.jit
def kernel(x, W):
    batch, qdim = x.shape
    edim = W.shape[0]
    grid = (batch // _ROW_BLOCK,)
    return pl.pallas_call(
        _topk_body,
        grid=grid,
        in_specs=[
            pl.BlockSpec((_ROW_BLOCK, qdim), lambda i: (i, 0)),
            pl.BlockSpec((edim, qdim), lambda i: (0, 0)),
        ],
        out_specs=pl.BlockSpec((_ROW_BLOCK, edim), lambda i: (i, 0)),
        out_shape=jax.ShapeDtypeStruct((batch, edim), jnp.float32),
    )(x, W)
